# Initial kernel scaffold; baseline (speedup 1.0000x reference)
import jax
import jax.numpy as jnp
from jax.experimental import pallas as pl


def kernel(user_embedding, item_embedding, edge_index, sub1_edge_index, sub2_edge_index, user_id, item_id, neg_item_id):
    raise NotImplementedError("write your pallas kernel here")


# speedup vs baseline: 13.9276x; 13.9276x over previous
import functools

import jax
import jax.numpy as jnp
from jax import lax
from jax.experimental import pallas as pl
from jax.experimental.pallas import tpu as pltpu
from jax.experimental.pallas import tpu_sc as plsc

_USER = 50000
_N = 100000
_D = 32
_HD = 16
_NP = 102400
_TBL = _NP
_PAD_ROW = _N
_CH = 128
_SUP = 8
_BN = 2048
_NB = _NP // _BN
_VB = 1000
_NVB = _USER // _VB
_TAU = 0.2
_BATCH = 1024

_f32 = jnp.float32
_i32 = jnp.int32


def _mesh():
    return plsc.VectorSubcoreMesh(core_axis_name="c", subcore_axis_name="s")


_SC_PARAMS = pltpu.CompilerParams(use_tc_tiling_on_sc=False)


@functools.cache
def _bincount_kernel(rows_total):
    rpc = rows_total // 2
    rps = rpc // 16
    nsup = rps // _SUP
    assert nsup * _SUP == rps and rpc * 2 == rows_total

    @functools.partial(
        pl.kernel,
        out_type=(jax.ShapeDtypeStruct((_NP, _HD), _f32),
                  jax.ShapeDtypeStruct((_NP, _HD), _f32)),
        mesh=_mesh(),
        compiler_params=_SC_PARAMS,
        scratch_types=(
            pltpu.VMEM((_SUP, _CH), _i32),
            pltpu.VMEM((_CH, _HD), _f32),
            pltpu.VMEM_SHARED((_TBL, _HD), _f32),
        ),
    )
    def bincount(src2, deg_a, deg_b, sidx, buf, table):
        c = lax.axis_index("c")
        s = lax.axis_index("s")

        def _fill(i, carry):
            buf[i, :] = jnp.zeros((_HD,), _f32)
            return carry
        lax.fori_loop(0, _CH, _fill, 0)

        zbase = s * (_TBL // 16)

        def _ztab(t, carry):
            pltpu.sync_copy(buf, table.at[pl.ds(zbase + t * _CH, _CH)])
            return carry
        lax.fori_loop(0, _TBL // 16 // _CH, _ztab, 0)

        def _ones(i, carry):
            buf[i, :] = jnp.ones((_HD,), _f32)
            return carry
        lax.fori_loop(0, _CH, _ones, 0)
        plsc.subcore_barrier()

        row0 = c * rpc + s * rps

        def _sup(u, carry):
            pltpu.sync_copy(src2.at[pl.ds(row0 + u * _SUP, _SUP)], sidx)

            def _j(j, cc):
                pltpu.sync_copy(buf, table.at[sidx.at[j]], add=True)
                return cc
            return lax.fori_loop(0, _SUP, _j, carry)
        lax.fori_loop(0, nsup, _sup, 0)

        plsc.subcore_barrier()
        obase = s * (_NP // 16)

        @pl.when(c == 0)
        def _():
            pltpu.sync_copy(table.at[pl.ds(obase, _NP // 16)],
                            deg_a.at[pl.ds(obase, _NP // 16)])

        @pl.when(c == 1)
        def _():
            pltpu.sync_copy(table.at[pl.ds(obase, _NP // 16)],
                            deg_b.at[pl.ds(obase, _NP // 16)])

    return bincount


@functools.cache
def _segsum_kernel(rows_total):
    rps = rows_total // 16
    nsup = rps // _SUP
    assert nsup * _SUP == rps

    @functools.partial(
        pl.kernel,
        out_type=(jax.ShapeDtypeStruct((_NP, _HD), _f32),
                  jax.ShapeDtypeStruct((_NP, _HD), _f32)),
        mesh=_mesh(),
        compiler_params=_SC_PARAMS,
        scratch_types=(
            pltpu.VMEM((_SUP, _CH), _i32),
            pltpu.VMEM((_SUP, _CH), _i32),
            pltpu.VMEM((_CH, _HD), _f32),
            pltpu.VMEM((_CH, _HD), _f32),
            pltpu.VMEM_SHARED((_TBL, _HD), _f32),
            pltpu.SemaphoreType.DMA,
        ),
    )
    def segsum(g_lo, g_hi, src2, dst2, agg_lo, agg_hi,
               sidx, didx, rows, zbuf, table, sem):
        c = lax.axis_index("c")
        s = lax.axis_index("s")

        def _fill(i, carry):
            zbuf[i, :] = jnp.zeros((_HD,), _f32)
            return carry
        lax.fori_loop(0, _CH, _fill, 0)

        zbase = s * (_TBL // 16)

        def _ztab(t, carry):
            pltpu.sync_copy(zbuf, table.at[pl.ds(zbase + t * _CH, _CH)])
            return carry
        lax.fori_loop(0, _TBL // 16 // _CH, _ztab, 0)
        plsc.subcore_barrier()

        row0 = s * rps

        def _run(g_ref):
            def _sup(u, carry):
                r = row0 + u * _SUP
                pltpu.sync_copy(src2.at[pl.ds(r, _SUP)], sidx)
                pltpu.sync_copy(dst2.at[pl.ds(r, _SUP)], didx)

                def _j(j, cc):
                    pltpu.async_copy(g_ref.at[sidx.at[j]], rows, sem).wait()
                    pltpu.sync_copy(rows, table.at[didx.at[j]], add=True)
                    return cc
                return lax.fori_loop(0, _SUP, _j, carry)
            lax.fori_loop(0, nsup, _sup, 0)

        @pl.when(c == 0)
        def _():
            _run(g_lo)

        @pl.when(c == 1)
        def _():
            _run(g_hi)

        plsc.subcore_barrier()
        obase = s * (_NP // 16)

        @pl.when(c == 0)
        def _():
            pltpu.sync_copy(table.at[pl.ds(obase, _NP // 16)],
                            agg_lo.at[pl.ds(obase, _NP // 16)])

        @pl.when(c == 1)
        def _():
            pltpu.sync_copy(table.at[pl.ds(obase, _NP // 16)],
                            agg_hi.at[pl.ds(obase, _NP // 16)])

    return segsum


@functools.partial(
    pl.kernel,
    out_type=tuple(jax.ShapeDtypeStruct((_BATCH, _D), _f32) for _ in range(7)),
    mesh=_mesh(),
    compiler_params=_SC_PARAMS,
    scratch_types=(
        pltpu.VMEM((32,), _i32),
        pltpu.VMEM((32, _D), _f32),
        pltpu.SemaphoreType.DMA,
    ),
)
def _gather7(acc_a, acc_b, acc_c, uid, iid, nid,
             ue, pe, ne, u1g, i1g, u2g, i2g, idxv, rows, sem):
    c = lax.axis_index("c")
    s = lax.axis_index("s")
    base = (s * 2 + c) * 32
    for tbl, idx, out in ((acc_a, uid, ue), (acc_a, iid, pe), (acc_a, nid, ne),
                          (acc_b, uid, u1g), (acc_b, iid, i1g),
                          (acc_c, uid, u2g), (acc_c, iid, i2g)):
        pltpu.sync_copy(idx.at[pl.ds(base, 32)], idxv)
        pltpu.async_copy(tbl.at[idxv], rows, sem).wait()
        pltpu.sync_copy(rows, out.at[pl.ds(base, 32)])


def _norm_scale_body(h0, daa, dab, dba, dbb, dca, dcb,
                     na, nb, nc, gal, gah, gbl, gbh, gcl, gch):
    h = h0[...]
    hl = h[:, :_HD]
    hh = h[:, _HD:]
    for da, db, no, gl, gh in ((daa, dab, na, gal, gah),
                               (dba, dbb, nb, gbl, gbh),
                               (dca, dcb, nc, gcl, gch)):
        d = da[...] + db[...]
        nrm = jnp.where(d > 0.0, lax.rsqrt(d), 0.0)
        no[...] = nrm
        gl[...] = hl * nrm
        gh[...] = hh * nrm


_bs16 = pl.BlockSpec((_BN, _HD), lambda i: (i, 0))
_bs32 = pl.BlockSpec((_BN, _D), lambda i: (i, 0))

_norm_scale = pl.pallas_call(
    _norm_scale_body,
    grid=(_NB,),
    in_specs=[_bs32] + [_bs16] * 6,
    out_specs=[_bs16] * 9,
    out_shape=[jax.ShapeDtypeStruct((_NP, _HD), _f32)] * 9,
)


def _post_body(al, ah, nm, acc, glo, gho, acco):
    nrm = nm[...]
    tl = al[...] * nrm
    th = ah[...] * nrm
    ss = (jnp.sum(tl * tl, axis=1, keepdims=True)
          + jnp.sum(th * th, axis=1, keepdims=True))
    inv = 1.0 / jnp.maximum(jnp.sqrt(ss), 1e-12)
    hl = tl * inv
    hh = th * inv
    acco[...] = acc[...] + jnp.concatenate([hl, hh], axis=1)
    glo[...] = hl * nrm
    gho[...] = hh * nrm


def _post_final_body(al, ah, nm, acc, acco):
    nrm = nm[...]
    tl = al[...] * nrm
    th = ah[...] * nrm
    ss = (jnp.sum(tl * tl, axis=1, keepdims=True)
          + jnp.sum(th * th, axis=1, keepdims=True))
    inv = 1.0 / jnp.maximum(jnp.sqrt(ss), 1e-12)
    acco[...] = acc[...] + jnp.concatenate([tl * inv, th * inv], axis=1)


_post = pl.pallas_call(
    _post_body,
    grid=(_NB,),
    in_specs=[_bs16, _bs16, _bs16, _bs32],
    out_specs=[_bs16, _bs16, _bs32],
    out_shape=[jax.ShapeDtypeStruct((_NP, _HD), _f32),
               jax.ShapeDtypeStruct((_NP, _HD), _f32),
               jax.ShapeDtypeStruct((_NP, _D), _f32)],
)

_post_final = pl.pallas_call(
    _post_final_body,
    grid=(_NB,),
    in_specs=[_bs16, _bs16, _bs16, _bs32],
    out_specs=[_bs32],
    out_shape=[jax.ShapeDtypeStruct((_NP, _D), _f32)],
)


def _nrm_rows(x):
    ss = jnp.sum(x * x, axis=1, keepdims=True)
    return x / jnp.maximum(jnp.sqrt(ss), 1e-12)


def _loss_body(ue, pe, ne, u1g, u2g, i1g, i2g, cu, ci, out, vu, vi):
    j = pl.program_id(0)
    u1n = _nrm_rows(u1g[...])
    i1n = _nrm_rows(i1g[...])
    au = _nrm_rows(cu[...])
    ai = _nrm_rows(ci[...])
    su = lax.dot_general(u1n, au, (((1,), (1,)), ((), ())),
                         preferred_element_type=_f32)
    pu = jnp.sum(jnp.exp(su / _TAU), axis=1, keepdims=True)
    si = lax.dot_general(i1n, ai, (((1,), (1,)), ((), ())),
                         preferred_element_type=_f32)
    pi = jnp.sum(jnp.exp(si / _TAU), axis=1, keepdims=True)

    @pl.when(j == 0)
    def _():
        vu[...] = pu
        vi[...] = pi

    @pl.when(j > 0)
    def _():
        vu[...] += pu
        vi[...] += pi

    @pl.when(j == _NVB - 1)
    def _():
        u2n = _nrm_rows(u2g[...])
        i2n = _nrm_rows(i2g[...])
        du = jnp.sum(u1n * u2n, axis=1) / _TAU
        di = jnp.sum(i1n * i2n, axis=1) / _TAU
        ssl_u = -jnp.sum(du - jnp.log(vu[...][:, 0]))
        ssl_i = -jnp.sum(di - jnp.log(vi[...][:, 0]))
        uee = ue[...] * 0.25
        pee = pe[...] * 0.25
        nee = ne[...] * 0.25
        x = jnp.sum(uee * pee, axis=1) - jnp.sum(uee * nee, axis=1)
        logsig = jnp.minimum(x, 0.0) - jnp.log(1.0 + jnp.exp(-jnp.abs(x)))
        bpr = -jnp.mean(logsig)
        reg = (jnp.sum(uee * uee) + jnp.sum(pee * pee)
               + jnp.sum(nee * nee)) * 0.5
        total = bpr + 1e-4 * reg / _BATCH + (ssl_u + ssl_i) * 0.1
        out[...] = jnp.reshape(total, (1, 1))


_bsb = pl.BlockSpec((_BATCH, _D), lambda j: (0, 0))

_loss = pl.pallas_call(
    _loss_body,
    grid=(_NVB,),
    in_specs=[_bsb] * 7 + [
        pl.BlockSpec((_VB, _D), lambda j: (j, 0)),
        pl.BlockSpec((_VB, _D), lambda j: (_USER // _VB + j, 0)),
    ],
    out_specs=pl.BlockSpec((1, 1), lambda j: (0, 0)),
    out_shape=jax.ShapeDtypeStruct((1, 1), _f32),
    scratch_shapes=[pltpu.VMEM((_BATCH, 1), _f32),
                    pltpu.VMEM((_BATCH, 1), _f32)],
)


def _prep_edges(ei):
    src = ei[0].astype(_i32)
    dst = ei[1].astype(_i32)
    e = src.shape[0]
    unit = 16 * _SUP * _CH * 2
    e_pad = ((e + unit - 1) // unit) * unit
    padn = e_pad - e
    srcg = jnp.concatenate([src, jnp.zeros((padn,), _i32)])
    srcs = jnp.concatenate([src, jnp.full((padn,), _PAD_ROW, _i32)])
    dsts = jnp.concatenate([dst, jnp.full((padn,), _PAD_ROW, _i32)])
    rt = e_pad // _CH
    return (srcg.reshape(rt, _CH), srcs.reshape(rt, _CH),
            dsts.reshape(rt, _CH), rt)


def kernel(user_embedding, item_embedding, edge_index, sub1_edge_index,
           sub2_edge_index, user_id, item_id, neg_item_id):
    h0 = jnp.concatenate([user_embedding, item_embedding,
                          jnp.zeros((_NP - _N, _D), _f32)], axis=0)

    ga = _prep_edges(edge_index)
    gb = _prep_edges(sub1_edge_index)
    gc = _prep_edges(sub2_edge_index)

    daa, dab = _bincount_kernel(ga[3])(ga[1])
    dba, dbb = _bincount_kernel(gb[3])(gb[1])
    dca, dcb = _bincount_kernel(gc[3])(gc[1])

    na, nb, nc, gal, gah, gbl, gbh, gcl, gch = _norm_scale(
        h0, daa, dab, dba, dbb, dca, dcb)

    def run_graph(prep, nrm, gl, gh, layers):
        srcg, _, dsts, rt = prep
        acc = h0
        for li in range(layers):
            al, ah = _segsum_kernel(rt)(gl, gh, srcg, dsts)
            if li == layers - 1:
                (acc,) = _post_final(al, ah, nrm, acc)
            else:
                gl, gh, acc = _post(al, ah, nrm, acc)
        return acc

    acc_a = run_graph(ga, na, gal, gah, 3)
    acc_b = run_graph(gb, nb, gbl, gbh, 2)
    acc_c = run_graph(gc, nc, gcl, gch, 2)

    uid = user_id.astype(_i32)
    iid = item_id.astype(_i32) + _USER
    nid = neg_item_id.astype(_i32) + _USER

    ue, pe, ne, u1g, i1g, u2g, i2g = _gather7(acc_a, acc_b, acc_c,
                                              uid, iid, nid)

    loss = _loss(ue, pe, ne, u1g, u2g, i1g, i2g, acc_c, acc_c)
    return loss[0, 0]
